# Initial kernel scaffold; baseline (speedup 1.0000x reference)
import jax
import jax.numpy as jnp
from jax.experimental import pallas as pl


def kernel(interaction, assessmentItemID, testId, KnowledgeTag, elapsed, difficulty, E_interaction, E_item, E_test, E_tag, W, b):
    raise NotImplementedError("write your pallas kernel here")


# speedup vs baseline: 5.3652x; 5.3652x over previous
import functools

import jax
import jax.numpy as jnp
from jax import lax
from jax.experimental import pallas as pl
from jax.experimental.pallas import tpu as pltpu
from jax.experimental.pallas import tpu_sc as plsc

B, S = 1024, 200
T = B * S
INTD, HD = 32, 96
IN_DIM = INTD * 4 + 2
WIDE = 4 * INTD

NC, NS = 2, 16
NW = NC * NS
T_PER_W = T // NW
CHUNK = 128
N_CHUNKS = T_PER_W // CHUNK

BT = 2048


def _sc_gather(tab_item, tab_test, tab_tag, idx_all, val_all):
    mesh = plsc.VectorSubcoreMesh(core_axis_name="c", subcore_axis_name="s")

    @functools.partial(
        pl.kernel,
        mesh=mesh,
        out_type=jax.ShapeDtypeStruct((T, WIDE), jnp.float32),
        scratch_types=[
            pltpu.VMEM((CHUNK,), jnp.int32),
            pltpu.VMEM((CHUNK,), jnp.int32),
            pltpu.VMEM((CHUNK,), jnp.int32),
            pltpu.VMEM((CHUNK,), jnp.int32),
            pltpu.VMEM((CHUNK,), jnp.float32),
            pltpu.VMEM((CHUNK,), jnp.float32),
            pltpu.VMEM((CHUNK, INTD), jnp.float32),
            pltpu.VMEM((CHUNK, INTD), jnp.float32),
            pltpu.VMEM((CHUNK, INTD), jnp.float32),
            pltpu.VMEM((CHUNK, 4), jnp.float32),
            pltpu.SemaphoreType.DMA,
        ],
        compiler_params=pltpu.CompilerParams(use_tc_tiling_on_sc=False,
                                             needs_layout_passes=False),
    )
    def k(tab1, tab2, tab3, idxs, vals, o,
          iv1, iv2, iv3, itv, ev, dv, rv1, rv2, rv3, s4, sem):
        wid = lax.axis_index("s") * NC + lax.axis_index("c")
        w_base = wid * T_PER_W

        def body(c, carry):
            base = w_base + c * CHUNK
            pltpu.sync_copy(idxs.at[pl.ds(base, CHUNK)], iv1)
            pltpu.sync_copy(idxs.at[pl.ds(T + base, CHUNK)], iv2)
            pltpu.sync_copy(idxs.at[pl.ds(2 * T + base, CHUNK)], iv3)
            pltpu.sync_copy(idxs.at[pl.ds(3 * T + base, CHUNK)], itv)
            pltpu.sync_copy(vals.at[pl.ds(base, CHUNK)], ev)
            pltpu.sync_copy(vals.at[pl.ds(T + base, CHUNK)], dv)
            c1 = pltpu.async_copy(tab1.at[iv1], rv1, sem)
            c2 = pltpu.async_copy(tab2.at[iv2], rv2, sem)
            c3 = pltpu.async_copy(tab3.at[iv3], rv3, sem)
            for g in range(CHUNK // 16):
                row = g * 16 + lax.iota(jnp.int32, 16)
                sl = pl.ds(g * 16, 16)
                plsc.store_scatter(s4, [row, jnp.zeros((16,), jnp.int32)],
                                   ev[sl])
                plsc.store_scatter(s4, [row, jnp.ones((16,), jnp.int32)],
                                   dv[sl])
                plsc.store_scatter(s4, [row, jnp.full((16,), 2, jnp.int32)],
                                   itv[sl].astype(jnp.float32))
            c1.wait()
            c2.wait()
            c3.wait()
            pltpu.sync_copy(rv1, o.at[pl.ds(base, CHUNK), pl.ds(0, INTD)])
            pltpu.sync_copy(rv2, o.at[pl.ds(base, CHUNK), pl.ds(INTD, INTD)])
            pltpu.sync_copy(rv3, o.at[pl.ds(base, CHUNK),
                                      pl.ds(2 * INTD, INTD)])
            pltpu.sync_copy(s4, o.at[pl.ds(base, CHUNK),
                                     pl.ds(3 * INTD, 4)])
            return carry

        lax.fori_loop(0, N_CHUNKS, body, 0)

    return k(tab_item, tab_test, tab_tag, idx_all, val_all)


def _tc_project(emb, E_int, W, b2):
    def mm(emb_ref, eint_ref, w_ref, b_ref, o_ref):
        buf = emb_ref[...]
        onehot = (buf[:, 3 * INTD + 2:3 * INTD + 3] ==
                  lax.broadcasted_iota(jnp.int32, (BT, 3), 1)
                  .astype(jnp.float32)).astype(jnp.float32)
        x_int = jnp.dot(onehot, eint_ref[...],
                        preferred_element_type=jnp.float32)
        acc = jnp.dot(x_int, w_ref[0:INTD, :],
                      preferred_element_type=jnp.float32)
        acc = acc + jnp.dot(buf[:, 0:3 * INTD + 2], w_ref[INTD:IN_DIM, :],
                            preferred_element_type=jnp.float32)
        o_ref[...] = acc + b_ref[...]

    tok = lambda i: (i, 0)
    rep = lambda i: (0, 0)
    return pl.pallas_call(
        mm,
        grid=(T // BT,),
        in_specs=[
            pl.BlockSpec((BT, WIDE), tok),
            pl.BlockSpec((3, INTD), rep),
            pl.BlockSpec((IN_DIM, HD), rep),
            pl.BlockSpec((1, HD), rep),
        ],
        out_specs=pl.BlockSpec((BT, HD), tok),
        out_shape=jax.ShapeDtypeStruct((T, HD), jnp.float32),
    )(emb, E_int, W, b2)


def kernel(interaction, assessmentItemID, testId, KnowledgeTag, elapsed,
           difficulty, E_interaction, E_item, E_test, E_tag, W, b):
    idx_all = jnp.concatenate([
        assessmentItemID.reshape(T).astype(jnp.int32),
        testId.reshape(T).astype(jnp.int32),
        KnowledgeTag.reshape(T).astype(jnp.int32),
        interaction.reshape(T).astype(jnp.int32),
    ])
    val_all = jnp.concatenate([elapsed.reshape(T), difficulty.reshape(T)])

    emb = _sc_gather(E_item, E_test, E_tag, idx_all, val_all)
    x = _tc_project(emb, E_interaction, W, b.reshape(1, HD))
    return x.reshape(B, S, HD)
